# baseline (speedup 1.0000x reference)
import functools

import jax
import jax.numpy as jnp
from jax.experimental import pallas as pl
from jax.experimental.pallas import tpu as pltpu

_T = 8192
_D = 4096
_E = 64
_TOP_K = 8
_BT = 256


def _router_body(x_ref, w_ref, wout_ref, iout_ref):
    x = x_ref[...]
    w = w_ref[...]
    logits = jnp.dot(x, w, preferred_element_type=jnp.float32)

    col = jax.lax.broadcasted_iota(jnp.int32, (_BT, _E), 1)
    vals = []
    idxs = []
    cur = logits
    for _ in range(_TOP_K):
        m = jnp.max(cur, axis=1, keepdims=True)
        is_max = cur == m
        sel = jnp.min(jnp.where(is_max, col, _E), axis=1, keepdims=True)
        vals.append(m)
        idxs.append(sel)
        cur = jnp.where(col == sel, -jnp.inf, cur)

    v = jnp.concatenate(vals, axis=1)
    e = jnp.exp(v - v[:, 0:1])
    wout_ref[...] = e / jnp.sum(e, axis=1, keepdims=True)
    iout_ref[...] = jnp.concatenate(idxs, axis=1).astype(jnp.int32)


@jax.jit
def kernel(x_TD, kernel_DE):
    x_TD = jnp.asarray(x_TD, jnp.float32)
    grid = (_T // _BT,)
    wout, iout = pl.pallas_call(
        _router_body,
        grid=grid,
        in_specs=[
            pl.BlockSpec((_BT, _D), lambda i: (i, 0)),
            pl.BlockSpec((_D, _E), lambda i: (0, 0)),
        ],
        out_specs=[
            pl.BlockSpec((_BT, _TOP_K), lambda i: (i, 0)),
            pl.BlockSpec((_BT, _TOP_K), lambda i: (i, 0)),
        ],
        out_shape=[
            jax.ShapeDtypeStruct((_T, _TOP_K), jnp.float32),
            jax.ShapeDtypeStruct((_T, _TOP_K), jnp.int32),
        ],
        compiler_params=pltpu.CompilerParams(
            dimension_semantics=("arbitrary",),
        ),
    )(x_TD, kernel_DE)
    return wout, iout


# speedup vs baseline: 1.1741x; 1.1741x over previous
import functools

import jax
import jax.numpy as jnp
from jax.experimental import pallas as pl
from jax.experimental.pallas import tpu as pltpu

_T = 8192
_D = 4096
_E = 64
_TOP_K = 8
_BT = 256


def _router_body(x_ref, w_ref, wout_ref, iout_ref):
    x = x_ref[...]
    w = w_ref[...]
    logits = jnp.dot(x, w, preferred_element_type=jnp.float32)

    coli = jax.lax.broadcasted_iota(jnp.int32, (_BT, _E), 1)
    negcol = (63 - coli).astype(jnp.float32)
    vals = []
    negsels = []
    cur = logits
    for _ in range(_TOP_K):
        m = jnp.max(cur, axis=1, keepdims=True)
        is_max = cur == m
        negsel = jnp.max(jnp.where(is_max, negcol, -1.0), axis=1, keepdims=True)
        vals.append(m)
        negsels.append(negsel)
        cur = jnp.where(negcol == negsel, -jnp.inf, cur)

    v = jnp.concatenate(vals, axis=1)
    e = jnp.exp(v - v[:, 0:1])
    wout_ref[...] = e / jnp.sum(e, axis=1, keepdims=True)
    idx_f = 63.0 - jnp.concatenate(negsels, axis=1)
    iout_ref[...] = idx_f.astype(jnp.int32)


@jax.jit
def kernel(x_TD, kernel_DE):
    x_TD = jnp.asarray(x_TD, jnp.float32)
    grid = (_T // _BT,)
    wout, iout = pl.pallas_call(
        _router_body,
        grid=grid,
        in_specs=[
            pl.BlockSpec((_BT, _D), lambda i: (i, 0)),
            pl.BlockSpec((_D, _E), lambda i: (0, 0)),
        ],
        out_specs=[
            pl.BlockSpec((_BT, _TOP_K), lambda i: (i, 0)),
            pl.BlockSpec((_BT, _TOP_K), lambda i: (i, 0)),
        ],
        out_shape=[
            jax.ShapeDtypeStruct((_T, _TOP_K), jnp.float32),
            jax.ShapeDtypeStruct((_T, _TOP_K), jnp.int32),
        ],
        compiler_params=pltpu.CompilerParams(
            dimension_semantics=("arbitrary",),
        ),
    )(x_TD, kernel_DE)
    return wout, iout


# speedup vs baseline: 1.1798x; 1.0048x over previous
import functools

import jax
import jax.numpy as jnp
from jax.experimental import pallas as pl
from jax.experimental.pallas import tpu as pltpu

_T = 8192
_D = 4096
_E = 64
_TOP_K = 8
_BT = 256


def _router_body(x_ref, w_ref, wout_ref, iout_ref):
    x = x_ref[...]
    w = w_ref[...]
    logits = jnp.dot(x, w, preferred_element_type=jnp.float32)

    coli = jax.lax.broadcasted_iota(jnp.int32, (_BT, _E), 1)
    negcol = (63 - coli).astype(jnp.float32)
    vals = []
    negsels = []
    cur = logits
    for _ in range(_TOP_K):
        m = jnp.max(cur, axis=1, keepdims=True)
        is_max = cur == m
        negsel = jnp.max(jnp.where(is_max, negcol, -1.0), axis=1, keepdims=True)
        vals.append(m)
        negsels.append(negsel)
        cur = jnp.where(negcol == negsel, -jnp.inf, cur)

    v = jnp.concatenate(vals, axis=1)
    e = jnp.exp(v - v[:, 0:1])
    wout_ref[...] = e / jnp.sum(e, axis=1, keepdims=True)
    idx_f = 63.0 - jnp.concatenate(negsels, axis=1)
    iout_ref[...] = idx_f.astype(jnp.int32)


@jax.jit
def kernel(x_TD, kernel_DE):
    x_TD = jnp.asarray(x_TD, jnp.float32)
    grid = (_T // _BT,)
    wout, iout = pl.pallas_call(
        _router_body,
        grid=grid,
        in_specs=[
            pl.BlockSpec((_BT, _D), lambda i: (i, 0)),
            pl.BlockSpec((_D, _E), lambda i: (0, 0)),
        ],
        out_specs=[
            pl.BlockSpec((_BT, _TOP_K), lambda i: (i, 0)),
            pl.BlockSpec((_BT, _TOP_K), lambda i: (i, 0)),
        ],
        out_shape=[
            jax.ShapeDtypeStruct((_T, _TOP_K), jnp.float32),
            jax.ShapeDtypeStruct((_T, _TOP_K), jnp.int32),
        ],
        compiler_params=pltpu.CompilerParams(
            dimension_semantics=("parallel",),
        ),
    )(x_TD, kernel_DE)
    return wout, iout
